# baseline (speedup 1.0000x reference)
import functools

import jax
import jax.numpy as jnp
from jax import lax
from jax.experimental import pallas as pl
from jax.experimental.pallas import tpu as pltpu
from jax.experimental.pallas import tpu_sc as plsc

N = 10000
E = 320000
D = 128

NC = 2
NS = 16
NW = NC * NS

SLEN = 80
NSUP = E // SLEN
NRB = 4
NIB = 6
GLAG = 3

RPT = (N // NS) // 8 * 8
TAIL = N - NS * RPT


def _agg_body(h_hbm, src_hbm, dst_hbm, out0, out1,
              sidx, didx, rows, accum, gsem, ssem, isem):
    c = lax.axis_index("c")
    s = lax.axis_index("s")
    wid = c * NS + s

    n = NSUP // NW
    lo = wid * n

    def issue_idx(it, buf):
        off = (lo + it) * SLEN
        pltpu.async_copy(src_hbm.at[pl.ds(off, SLEN)], sidx.at[buf], isem)
        pltpu.async_copy(dst_hbm.at[pl.ds(off, SLEN)], didx.at[buf], isem)

    issue_idx(0, 0)
    issue_idx(1, 1)
    z16 = jnp.zeros((16,), jnp.float32)
    for r in range(SLEN):
        for cc in range(D // 16):
            rows[0, r, pl.ds(cc * 16, 16)] = z16
    sl = pl.ds(s * RPT, RPT)
    tl = pl.ds(NS * RPT, TAIL)
    last = s == NS - 1
    for k in range(RPT // SLEN):
        pltpu.sync_copy(rows.at[0],
                        accum.at[pl.ds(s * RPT + k * SLEN, SLEN)])
    rem = RPT % SLEN
    pltpu.sync_copy(rows.at[0, pl.ds(0, rem)],
                    accum.at[pl.ds(s * RPT + RPT - rem, rem)])

    @pl.when(last)
    def _():
        pltpu.sync_copy(rows.at[0, pl.ds(0, TAIL)], accum.at[tl])

    plsc.subcore_barrier()

    def gissue(j):
        pltpu.async_copy(h_hbm.at[sidx.at[j % NIB]], rows.at[j % NRB], gsem)

    def gwait(j):
        pltpu.make_async_copy(h_hbm.at[sidx.at[j % NIB]],
                              rows.at[j % NRB], gsem).wait()

    def sissue(j):
        pltpu.async_copy(rows.at[j % NRB], accum.at[didx.at[j % NIB]],
                         ssem, add=True)

    def swait(j):
        pltpu.make_async_copy(rows.at[j % NRB],
                              accum.at[didx.at[j % NIB]], ssem).wait()

    def body(i, _):
        @pl.when(i >= NRB)
        def _():
            swait(i - NRB)

        pltpu.make_async_copy(src_hbm.at[pl.ds(0, SLEN)],
                              sidx.at[i % NIB], isem).wait()
        pltpu.make_async_copy(dst_hbm.at[pl.ds(0, SLEN)],
                              didx.at[i % NIB], isem).wait()

        @pl.when(i + 2 < n)
        def _():
            issue_idx(i + 2, (i + 2) % NIB)

        gissue(i)

        @pl.when(i >= GLAG)
        def _():
            gwait(i - GLAG)
            sissue(i - GLAG)
        return 0

    lax.fori_loop(0, n, body, 0)

    for k in range(GLAG):
        j = n - GLAG + k
        gwait(j)
        sissue(j)
    for k in range(NRB):
        swait(n - NRB + k)
    plsc.subcore_barrier()

    @pl.when(c == 0)
    def _():
        pltpu.sync_copy(accum.at[sl], out0.at[sl])

        @pl.when(last)
        def _():
            pltpu.sync_copy(accum.at[tl], out0.at[tl])

    @pl.when(c == 1)
    def _():
        pltpu.sync_copy(accum.at[sl], out1.at[sl])

        @pl.when(last)
        def _():
            pltpu.sync_copy(accum.at[tl], out1.at[tl])


@functools.cache
def _make_agg():
    return pl.kernel(
        _agg_body,
        out_type=(
            jax.ShapeDtypeStruct((N, D), jnp.float32),
            jax.ShapeDtypeStruct((N, D), jnp.float32),
        ),
        mesh=plsc.VectorSubcoreMesh(core_axis_name="c", subcore_axis_name="s",
                                    num_cores=NC, num_subcores=NS),
        scratch_types=[
            pltpu.VMEM((NIB, SLEN), jnp.int32),
            pltpu.VMEM((NIB, SLEN), jnp.int32),
            pltpu.VMEM((NRB, SLEN, D), jnp.float32),
            pltpu.VMEM_SHARED((N, D), jnp.float32),
            pltpu.SemaphoreType.DMA,
            pltpu.SemaphoreType.DMA,
            pltpu.SemaphoreType.DMA,
        ],
    )


def _mlp_body(relu_out, h_ref, p0_ref, p1_ref, w1_ref, b1_ref, w2_ref,
              b2_ref, o_ref):
    x = h_ref[...] + (p0_ref[...] + p1_ref[...])
    z = jnp.dot(x, w1_ref[...],
                preferred_element_type=jnp.float32) + b1_ref[...]
    z = jnp.maximum(z, 0.0)
    y = jnp.dot(z, w2_ref[...],
                preferred_element_type=jnp.float32) + b2_ref[...]
    if relu_out:
        y = jnp.maximum(y, 0.0)
    o_ref[...] = y


def _mlp(h, p0, p1, w1, b1, w2, b2, relu_out):
    blk = 5000
    grid = (N // blk,)
    row_spec = pl.BlockSpec((blk, D), lambda i: (i, 0))
    full_spec = pl.BlockSpec((D, D), lambda i: (0, 0))
    bias_spec = pl.BlockSpec((1, D), lambda i: (0, 0))
    return pl.pallas_call(
        functools.partial(_mlp_body, relu_out),
        grid=grid,
        in_specs=[row_spec, row_spec, row_spec, full_spec, bias_spec,
                  full_spec, bias_spec],
        out_specs=row_spec,
        out_shape=jax.ShapeDtypeStruct((N, D), jnp.float32),
        compiler_params=pltpu.CompilerParams(
            dimension_semantics=("parallel",),
        ),
    )(h, p0, p1, w1, b1.reshape(1, D), w2, b2.reshape(1, D))


def kernel(x, edge_index, W1_1, b1_1, W2_1, b2_1, W1_2, b1_2, W2_2, b2_2,
           W1_3, b1_3, W2_3, b2_3):
    src = edge_index[0]
    dst = edge_index[1]

    h = x
    for w1, b1, w2, b2, relu_out in (
        (W1_1, b1_1, W2_1, b2_1, True),
        (W1_2, b1_2, W2_2, b2_2, True),
        (W1_3, b1_3, W2_3, b2_3, False),
    ):
        p0, p1 = _make_agg()(h, src, dst)
        h = _mlp(h, p0, p1, w1, b1, w2, b2, relu_out)
    return h
